# baseline (speedup 1.0000x reference)
import functools

import jax
import jax.numpy as jnp
from jax import lax
from jax.experimental import pallas as pl
from jax.experimental.pallas import tpu as pltpu
from jax.experimental.pallas import tpu_sc as plsc

N_GRAPHS = 512
ROWS_PER_BLOCK = 7168
SUB_ROWS = ROWS_PER_BLOCK // 8
N_WORKERS = 16
CHUNKS = 49
BIN_PAD = 528


def _tc_dot_body(x_ref, w_ref, o_ref):
    w = w_ref[...]
    for j in range(8):
        s = jax.lax.dot_general(w, x_ref[pl.ds(j * SUB_ROWS, SUB_ROWS), :],
                                (((1,), (1,)), ((), ())),
                                preferred_element_type=jnp.float32)
        o_ref[0, j, :] = s[0]


def _sc_segment_sum(y2, b1, n_nodes):
    mesh = plsc.VectorSubcoreMesh(core_axis_name="c", subcore_axis_name="s",
                                  num_cores=2, num_subcores=16)
    n_work = 14
    rows_w = 8
    chunks_w = rows_w * 896 // 128
    tail_n = n_nodes - (n_work - 1) * rows_w * 896
    tail_full = tail_n // 128
    tail_rem = tail_n - tail_full * 128

    @functools.partial(
        pl.kernel,
        out_type=jax.ShapeDtypeStruct((N_GRAPHS,), jnp.float32),
        mesh=mesh,
        scratch_types=[
            pltpu.VMEM((rows_w, 896), jnp.float32),
            pltpu.VMEM((chunks_w, 128), jnp.int32),
            pltpu.VMEM((BIN_PAD,), jnp.float32),
            pltpu.VMEM((BIN_PAD // 128 + 1, 128), jnp.int32),
            pltpu.VMEM_SHARED((BIN_PAD,), jnp.float32),
            pltpu.SemaphoreType.DMA,
            pltpu.SemaphoreType.DMA,
        ],
        compiler_params=pltpu.CompilerParams(needs_layout_passes=False),
    )
    def seg_sum(y_hbm, b_hbm, out_hbm, val_v, idx_v, binsl_v, id_v, bins_sh,
                sem_in, sem_sc):
        c = lax.axis_index("c")
        s = lax.axis_index("s")

        def load(n_full, rem):
            cps = [pltpu.async_copy(y_hbm.at[pl.ds(s * rows_w, rows_w)],
                                    val_v, sem_in)]
            base = s * rows_w * 896
            for j in range(n_full):
                cps.append(pltpu.async_copy(
                    b_hbm.at[pl.ds(base + 128 * j, 128)],
                    idx_v.at[j], sem_in))
            if rem:
                cps.append(pltpu.async_copy(
                    b_hbm.at[pl.ds(base + 128 * n_full, rem)],
                    idx_v.at[n_full, pl.ds(0, rem)], sem_in))
            for cp in cps:
                cp.wait()

        def accumulate(n_chunks):
            for g in range(n_chunks * 8):
                idx16 = idx_v[g // 8, pl.ds(16 * (g % 8), 16)]
                val16 = val_v[g // 56, pl.ds(16 * (g % 56), 16)]
                plsc.addupdate_scatter(binsl_v, [idx16], val16)

        @pl.when(c == 0)
        def _core0():
            for k in range(BIN_PAD // 16):
                binsl_v[pl.ds(16 * k, 16)] = jnp.zeros((16,), jnp.float32)
            iota16 = lax.iota(jnp.int32, 16)
            for k in range(BIN_PAD // 16):
                id_v[k // 8, pl.ds(16 * (k % 8), 16)] = iota16 + 16 * k

            @pl.when(s == 0)
            def _zero_shared():
                pltpu.sync_copy(binsl_v, bins_sh)

            @pl.when(s < n_work - 1)
            def _load_a():
                load(chunks_w, 0)

            @pl.when(s == n_work - 1)
            def _load_tail():
                for j in range(tail_rem // 16, 8):
                    idx_v[tail_full, pl.ds(16 * j, 16)] = jnp.full(
                        (16,), N_GRAPHS, jnp.int32)
                load(tail_full, tail_rem)

            @pl.when(s < n_work - 1)
            def _acc_a():
                accumulate(chunks_w)

            @pl.when(s == n_work - 1)
            def _acc_tail():
                accumulate(tail_full + 1)

            plsc.subcore_barrier()

            cps = []
            for k in range(BIN_PAD // 128):
                cps.append(pltpu.async_copy(
                    binsl_v.at[pl.ds(128 * k, 128)],
                    bins_sh.at[id_v.at[k]], sem_sc, add=True))
            rem_b = BIN_PAD % 128
            if rem_b:
                cps.append(pltpu.async_copy(
                    binsl_v.at[pl.ds(BIN_PAD - rem_b, rem_b)],
                    bins_sh.at[id_v.at[BIN_PAD // 128, pl.ds(0, rem_b)]],
                    sem_sc, add=True))
            for cp in cps:
                cp.wait()

            plsc.subcore_barrier()

            @pl.when(s == 0)
            def _write_out():
                pltpu.sync_copy(bins_sh.at[pl.ds(0, N_GRAPHS)], out_hbm)

    return seg_sum(y2, b1)


def kernel(node_features, batch, W):
    n, d = node_features.shape
    n_blocks = -(-n // ROWS_PER_BLOCK)
    n_pad = n_blocks * ROWS_PER_BLOCK

    y3 = pl.pallas_call(
        _tc_dot_body,
        grid=(n_blocks,),
        in_specs=[
            pl.BlockSpec((ROWS_PER_BLOCK, d), lambda i: (i, 0)),
            pl.BlockSpec((1, d), lambda i: (0, 0)),
        ],
        out_specs=pl.BlockSpec((1, 8, SUB_ROWS), lambda i: (i, 0, 0)),
        out_shape=jax.ShapeDtypeStruct((n_blocks, 8, SUB_ROWS), jnp.float32),
    )(node_features, W.reshape(1, d))
    y2 = y3.reshape(n_blocks * 8, SUB_ROWS)

    out = _sc_segment_sum(y2, batch.astype(jnp.int32), n)
    return out.reshape(N_GRAPHS, 1)


# speedup vs baseline: 1.0891x; 1.0891x over previous
import functools

import jax
import jax.numpy as jnp
from jax import lax
from jax.experimental import pallas as pl
from jax.experimental.pallas import tpu as pltpu
from jax.experimental.pallas import tpu_sc as plsc

N_GRAPHS = 512
ROWS_PER_BLOCK = 7168
SUB_ROWS = ROWS_PER_BLOCK // 8
N_WORKERS = 16
CHUNKS = 49
BIN_PAD = 528


def _tc_dot_body(x_ref, w_ref, o_ref):
    w = w_ref[...]
    for j in range(8):
        s = jax.lax.dot_general(w, x_ref[pl.ds(j * SUB_ROWS, SUB_ROWS), :],
                                (((1,), (1,)), ((), ())),
                                preferred_element_type=jnp.float32)
        o_ref[0, j, :] = s[0]


def _sc_segment_sum(y2, b1, n_nodes):
    mesh = plsc.VectorSubcoreMesh(core_axis_name="c", subcore_axis_name="s",
                                  num_cores=2, num_subcores=16)
    n_work = 14
    rows_w = 8
    chunks_w = rows_w * 896 // 128
    tail_n = n_nodes - (n_work - 1) * rows_w * 896
    tail_full = tail_n // 128
    tail_rem = tail_n - tail_full * 128

    @functools.partial(
        pl.kernel,
        out_type=jax.ShapeDtypeStruct((N_GRAPHS,), jnp.float32),
        mesh=mesh,
        scratch_types=[
            pltpu.VMEM((rows_w, 896), jnp.float32),
            pltpu.VMEM((chunks_w, 128), jnp.int32),
            pltpu.VMEM((BIN_PAD,), jnp.float32),
            pltpu.VMEM_SHARED((BIN_PAD,), jnp.float32),
            pltpu.SemaphoreType.DMA,
            pltpu.SemaphoreType.DMA,
        ],
        compiler_params=pltpu.CompilerParams(needs_layout_passes=False),
    )
    def seg_sum(y_hbm, b_hbm, out_hbm, val_v, idx_v, zbuf_v, bins_sh,
                sem_in, sem_sc):
        c = lax.axis_index("c")
        s = lax.axis_index("s")

        def load(n_full, rem):
            cps = [pltpu.async_copy(y_hbm.at[pl.ds(s * rows_w, rows_w)],
                                    val_v, sem_in)]
            base = s * rows_w * 896
            for j in range(n_full):
                cps.append(pltpu.async_copy(
                    b_hbm.at[pl.ds(base + 128 * j, 128)],
                    idx_v.at[j], sem_in))
            if rem:
                cps.append(pltpu.async_copy(
                    b_hbm.at[pl.ds(base + 128 * n_full, rem)],
                    idx_v.at[n_full, pl.ds(0, rem)], sem_in))
            for cp in cps:
                cp.wait()

        def accumulate(n_chunks):
            cps = [pltpu.async_copy(
                       val_v.at[j // 7, pl.ds(128 * (j % 7), 128)],
                       bins_sh.at[idx_v.at[j]], sem_sc, add=True)
                   for j in range(n_chunks)]
            for cp in cps:
                cp.wait()

        @pl.when(c == 0)
        def _core0():
            @pl.when(s == 0)
            def _zero_shared():
                for k in range(BIN_PAD // 16):
                    zbuf_v[pl.ds(16 * k, 16)] = jnp.zeros((16,), jnp.float32)
                pltpu.sync_copy(zbuf_v, bins_sh)

            @pl.when(s < n_work - 1)
            def _load_a():
                load(chunks_w, 0)

            @pl.when(s == n_work - 1)
            def _load_tail():
                for j in range(tail_rem // 16, 8):
                    idx_v[tail_full, pl.ds(16 * j, 16)] = jnp.full(
                        (16,), N_GRAPHS, jnp.int32)
                load(tail_full, tail_rem)

            plsc.subcore_barrier()

            @pl.when(s < n_work - 1)
            def _acc_a():
                accumulate(chunks_w)

            @pl.when(s == n_work - 1)
            def _acc_tail():
                accumulate(tail_full + 1)

            plsc.subcore_barrier()

            @pl.when(s == 0)
            def _write_out():
                pltpu.sync_copy(bins_sh.at[pl.ds(0, N_GRAPHS)], out_hbm)

    return seg_sum(y2, b1)


def kernel(node_features, batch, W):
    n, d = node_features.shape
    n_blocks = -(-n // ROWS_PER_BLOCK)
    n_pad = n_blocks * ROWS_PER_BLOCK

    y3 = pl.pallas_call(
        _tc_dot_body,
        grid=(n_blocks,),
        in_specs=[
            pl.BlockSpec((ROWS_PER_BLOCK, d), lambda i: (i, 0)),
            pl.BlockSpec((1, d), lambda i: (0, 0)),
        ],
        out_specs=pl.BlockSpec((1, 8, SUB_ROWS), lambda i: (i, 0, 0)),
        out_shape=jax.ShapeDtypeStruct((n_blocks, 8, SUB_ROWS), jnp.float32),
    )(node_features, W.reshape(1, d))
    y2 = y3.reshape(n_blocks * 8, SUB_ROWS)

    out = _sc_segment_sum(y2, batch.astype(jnp.int32), n)
    return out.reshape(N_GRAPHS, 1)
